# baseline (speedup 1.0000x reference)
import jax
import jax.numpy as jnp
from jax import lax
from jax.experimental import pallas as pl
from jax.experimental.pallas import tpu as pltpu

_Q = 1024
_D = 16
_K = 100000
_KB = 4096
_NB = (_K + _KB - 1) // _KB
_KPAD = _NB * _KB
_TOPK = 5
_LANES = 128
_NCH = _KB // _LANES
_CAND = _TOPK * _LANES
_PAD_VAL = 1.0e6


def _fold_kernel(xp_ref, xt_ref, s_ref):
    i = pl.program_id(0)

    @pl.when(i == 0)
    def _init():
        s_ref[...] = jnp.full((_Q, _CAND), jnp.inf, dtype=jnp.float32)

    xp = xp_ref[...]
    xt = xt_ref[...]
    sq_p = jnp.sum(xp * xp, axis=1, keepdims=True)
    sq_t = jnp.sum(xt * xt, axis=1, keepdims=True)
    xp_aug = jnp.concatenate(
        [xp, sq_p, jnp.ones((_Q, 1), jnp.float32)], axis=1)
    xt_aug = jnp.concatenate(
        [-2.0 * xt, jnp.ones((_KB, 1), jnp.float32), sq_t], axis=1)
    d2 = lax.dot_general(xp_aug, xt_aug, (((1,), (1,)), ((), ())),
                         preferred_element_type=jnp.float32)

    s = [s_ref[:, j * _LANES:(j + 1) * _LANES] for j in range(_TOPK)]
    for c in range(_NCH):
        t = d2[:, c * _LANES:(c + 1) * _LANES]
        for j in range(_TOPK):
            lo = jnp.minimum(s[j], t)
            if j < _TOPK - 1:
                t = jnp.maximum(s[j], t)
            s[j] = lo
    for j in range(_TOPK):
        s_ref[:, j * _LANES:(j + 1) * _LANES] = s[j]


def _finish_kernel(s_ref, out_ref):
    cand = s_ref[...]
    iota = lax.broadcasted_iota(jnp.int32, (_Q, _CAND), 1)
    vals = []
    for _ in range(_TOPK):
        m = jnp.min(cand, axis=1, keepdims=True)
        vals.append(m)
        hit = jnp.where(cand <= m, iota, _CAND)
        first = jnp.min(hit, axis=1, keepdims=True)
        cand = jnp.where(iota == first, jnp.inf, cand)
    d = jnp.sqrt(jnp.maximum(jnp.concatenate(vals, axis=1), 0.0))
    hinged = jnp.maximum(d - 1.0, 0.0)
    out_ref[...] = (jnp.sum(hinged) / (_Q * _TOPK)).reshape(1, 1)


def kernel(x_pred, x_target, top_k):
    xt_pad = jnp.pad(x_target, ((0, _KPAD - _K), (0, 0)),
                     constant_values=_PAD_VAL)
    survivors = pl.pallas_call(
        _fold_kernel,
        grid=(_NB,),
        in_specs=[
            pl.BlockSpec((_Q, _D), lambda i: (0, 0)),
            pl.BlockSpec((_KB, _D), lambda i: (i, 0)),
        ],
        out_specs=pl.BlockSpec((_Q, _CAND), lambda i: (0, 0)),
        out_shape=jax.ShapeDtypeStruct((_Q, _CAND), jnp.float32),
        compiler_params=pltpu.CompilerParams(
            dimension_semantics=("arbitrary",)),
    )(x_pred, xt_pad)
    out = pl.pallas_call(
        _finish_kernel,
        out_shape=jax.ShapeDtypeStruct((1, 1), jnp.float32),
    )(survivors)
    return out[0, 0] + 0.0 * top_k


# speedup vs baseline: 1.0557x; 1.0557x over previous
import jax
import jax.numpy as jnp
from jax import lax
from jax.experimental import pallas as pl
from jax.experimental.pallas import tpu as pltpu

_Q = 1024
_D = 16
_K = 100000
_KB = 4096
_NB = (_K + _KB - 1) // _KB
_KPAD = _NB * _KB
_TOPK = 5
_LANES = 128
_NCH = _KB // _LANES
_CAND = _TOPK * _LANES
_PAD_VAL = 1.0e6

_SORT8_BOTTOM5 = [
    (0, 1, True), (2, 3, True), (4, 5, True), (6, 7, True),
    (0, 2, True), (1, 3, True), (4, 6, True), (5, 7, True),
    (1, 2, True), (5, 6, True),
    (0, 4, True), (1, 5, True), (2, 6, False), (3, 7, False),
    (2, 4, True), (3, 5, False),
    (1, 2, True), (3, 4, True),
]


def _fold_kernel(xp_ref, xt_ref, s_ref):
    i = pl.program_id(0)

    @pl.when(i == 0)
    def _init():
        s_ref[...] = jnp.full((_Q, _CAND), jnp.inf, dtype=jnp.float32)

    xp = xp_ref[...]
    xt = xt_ref[...]
    sq_p = jnp.sum(xp * xp, axis=1, keepdims=True)
    sq_t = jnp.sum(xt * xt, axis=1, keepdims=True)
    xp_aug = jnp.concatenate(
        [xp, sq_p, jnp.ones((_Q, 1), jnp.float32)], axis=1)
    xt_aug = jnp.concatenate(
        [-2.0 * xt, jnp.ones((_KB, 1), jnp.float32), sq_t], axis=1)
    d2 = lax.dot_general(xp_aug, xt_aug, (((1,), (1,)), ((), ())),
                         preferred_element_type=jnp.float32)

    s = [s_ref[:, j * _LANES:(j + 1) * _LANES] for j in range(_TOPK)]
    for g in range(_NCH // 8):
        v = [d2[:, (8 * g + c) * _LANES:(8 * g + c + 1) * _LANES]
             for c in range(8)]
        for i, j, need_max in _SORT8_BOTTOM5:
            lo = jnp.minimum(v[i], v[j])
            if need_max:
                v[j] = jnp.maximum(v[i], v[j])
            v[i] = lo
        for m in range(_TOPK):
            t = v[m]
            for j in range(m, _TOPK):
                lo = jnp.minimum(s[j], t)
                if j < _TOPK - 1:
                    t = jnp.maximum(s[j], t)
                s[j] = lo
    for j in range(_TOPK):
        s_ref[:, j * _LANES:(j + 1) * _LANES] = s[j]


def _finish_kernel(s_ref, out_ref):
    cand = s_ref[...]
    iota = lax.broadcasted_iota(jnp.int32, (_Q, _CAND), 1)
    vals = []
    for _ in range(_TOPK):
        m = jnp.min(cand, axis=1, keepdims=True)
        vals.append(m)
        hit = jnp.where(cand <= m, iota, _CAND)
        first = jnp.min(hit, axis=1, keepdims=True)
        cand = jnp.where(iota == first, jnp.inf, cand)
    d = jnp.sqrt(jnp.maximum(jnp.concatenate(vals, axis=1), 0.0))
    hinged = jnp.maximum(d - 1.0, 0.0)
    out_ref[...] = (jnp.sum(hinged) / (_Q * _TOPK)).reshape(1, 1)


def kernel(x_pred, x_target, top_k):
    xt_pad = jnp.pad(x_target, ((0, _KPAD - _K), (0, 0)),
                     constant_values=_PAD_VAL)
    survivors = pl.pallas_call(
        _fold_kernel,
        grid=(_NB,),
        in_specs=[
            pl.BlockSpec((_Q, _D), lambda i: (0, 0)),
            pl.BlockSpec((_KB, _D), lambda i: (i, 0)),
        ],
        out_specs=pl.BlockSpec((_Q, _CAND), lambda i: (0, 0)),
        out_shape=jax.ShapeDtypeStruct((_Q, _CAND), jnp.float32),
        compiler_params=pltpu.CompilerParams(
            dimension_semantics=("arbitrary",)),
    )(x_pred, xt_pad)
    out = pl.pallas_call(
        _finish_kernel,
        out_shape=jax.ShapeDtypeStruct((1, 1), jnp.float32),
    )(survivors)
    return out[0, 0] + 0.0 * top_k


# speedup vs baseline: 1.2579x; 1.1916x over previous
import jax
import jax.numpy as jnp
from jax import lax
from jax.experimental import pallas as pl
from jax.experimental.pallas import tpu as pltpu

_Q = 1024
_D = 16
_K = 100000
_KB = 4096
_NBF = 24
_TAIL = _K - _NBF * _KB
_TPAD = 2048
_TOPK = 5
_LANES = 128
_CAND = _TOPK * _LANES
_PAD_VAL = 1.0e6

_SORT8_BOTTOM5 = [
    (0, 1, True), (2, 3, True), (4, 5, True), (6, 7, True),
    (0, 2, True), (1, 3, True), (4, 6, True), (5, 7, True),
    (1, 2, True), (5, 6, True),
    (0, 4, True), (1, 5, True), (2, 6, False), (3, 7, False),
    (2, 4, True), (3, 5, False),
    (1, 2, True), (3, 4, True),
]


def _dist2(xp, xt, width):
    sq_p = jnp.sum(xp * xp, axis=1, keepdims=True)
    sq_t = jnp.sum(xt * xt, axis=1, keepdims=True)
    xp_aug = jnp.concatenate(
        [xp, sq_p, jnp.ones((_Q, 1), jnp.float32)], axis=1)
    xt_aug = jnp.concatenate(
        [-2.0 * xt, jnp.ones((width, 1), jnp.float32), sq_t], axis=1)
    return lax.dot_general(xp_aug, xt_aug, (((1,), (1,)), ((), ())),
                           preferred_element_type=jnp.float32)


def _fold(s_ref, d2, nch):
    s = [s_ref[:, j * _LANES:(j + 1) * _LANES] for j in range(_TOPK)]
    for g in range(nch // 8):
        v = [d2[:, (8 * g + c) * _LANES:(8 * g + c + 1) * _LANES]
             for c in range(8)]
        for i, j, need_max in _SORT8_BOTTOM5:
            lo = jnp.minimum(v[i], v[j])
            if need_max:
                v[j] = jnp.maximum(v[i], v[j])
            v[i] = lo
        for m in range(_TOPK):
            t = v[m]
            for j in range(m, _TOPK):
                lo = jnp.minimum(s[j], t)
                if j < _TOPK - 1:
                    t = jnp.maximum(s[j], t)
                s[j] = lo
    for j in range(_TOPK):
        s_ref[:, j * _LANES:(j + 1) * _LANES] = s[j]


def _loss_kernel(xp_ref, xt_ref, xtt_ref, out_ref, s_ref):
    i = pl.program_id(0)

    @pl.when(i == 0)
    def _init():
        s_ref[...] = jnp.full((_Q, _CAND), jnp.inf, dtype=jnp.float32)

    xp = xp_ref[...]

    @pl.when(i < _NBF)
    def _main():
        _fold(s_ref, _dist2(xp, xt_ref[...], _KB), _KB // _LANES)

    @pl.when(i == _NBF)
    def _finish():
        _fold(s_ref, _dist2(xp, xtt_ref[...], _TPAD), _TPAD // _LANES)
        cand = s_ref[...]
        iota = lax.broadcasted_iota(jnp.int32, (_Q, _CAND), 1)
        vals = []
        for _ in range(_TOPK):
            m = jnp.min(cand, axis=1, keepdims=True)
            vals.append(m)
            hit = jnp.where(cand <= m, iota, _CAND)
            first = jnp.min(hit, axis=1, keepdims=True)
            cand = jnp.where(iota == first, jnp.inf, cand)
        d = jnp.sqrt(jnp.maximum(jnp.concatenate(vals, axis=1), 0.0))
        hinged = jnp.maximum(d - 1.0, 0.0)
        out_ref[...] = (jnp.sum(hinged) / (_Q * _TOPK)).reshape(1, 1)


def kernel(x_pred, x_target, top_k):
    xt_tail = jnp.pad(x_target[_NBF * _KB:], ((0, _TPAD - _TAIL), (0, 0)),
                      constant_values=_PAD_VAL)
    out = pl.pallas_call(
        _loss_kernel,
        grid=(_NBF + 1,),
        in_specs=[
            pl.BlockSpec((_Q, _D), lambda i: (0, 0)),
            pl.BlockSpec((_KB, _D), lambda i: (jnp.minimum(i, _NBF - 1), 0)),
            pl.BlockSpec((_TPAD, _D), lambda i: (0, 0)),
        ],
        out_specs=pl.BlockSpec((1, 1), lambda i: (0, 0)),
        out_shape=jax.ShapeDtypeStruct((1, 1), jnp.float32),
        scratch_shapes=[pltpu.VMEM((_Q, _CAND), jnp.float32)],
        compiler_params=pltpu.CompilerParams(
            dimension_semantics=("arbitrary",)),
    )(x_pred, x_target, xt_tail)
    return out[0, 0] + 0.0 * top_k


# speedup vs baseline: 1.2689x; 1.0087x over previous
import jax
import jax.numpy as jnp
from jax import lax
from jax.experimental import pallas as pl
from jax.experimental.pallas import tpu as pltpu

_Q = 1024
_D = 16
_K = 100000
_KB = 4096
_NBF = 24
_TAIL = _K - _NBF * _KB
_TPAD = 2048
_TBLK = _NBF * _KB // _TPAD
_TOPK = 5
_LANES = 128
_CAND = _TOPK * _LANES

_SORT8_BOTTOM5 = [
    (0, 1, True), (2, 3, True), (4, 5, True), (6, 7, True),
    (0, 2, True), (1, 3, True), (4, 6, True), (5, 7, True),
    (1, 2, True), (5, 6, True),
    (0, 4, True), (1, 5, True), (2, 6, False), (3, 7, False),
    (2, 4, True), (3, 5, False),
    (1, 2, True), (3, 4, True),
]


def _dist2(xp, xt, width):
    sq_p = jnp.sum(xp * xp, axis=1, keepdims=True)
    sq_t = jnp.sum(xt * xt, axis=1, keepdims=True)
    xp_aug = jnp.concatenate(
        [xp, sq_p, jnp.ones((_Q, 1), jnp.float32)], axis=1)
    xt_aug = jnp.concatenate(
        [-2.0 * xt, jnp.ones((width, 1), jnp.float32), sq_t], axis=1)
    return lax.dot_general(xp_aug, xt_aug, (((1,), (1,)), ((), ())),
                           preferred_element_type=jnp.float32)


def _fold(s_ref, d2, nch):
    s = [s_ref[:, j * _LANES:(j + 1) * _LANES] for j in range(_TOPK)]
    for g in range(nch // 8):
        v = [d2[:, (8 * g + c) * _LANES:(8 * g + c + 1) * _LANES]
             for c in range(8)]
        for i, j, need_max in _SORT8_BOTTOM5:
            lo = jnp.minimum(v[i], v[j])
            if need_max:
                v[j] = jnp.maximum(v[i], v[j])
            v[i] = lo
        for m in range(_TOPK):
            t = v[m]
            for j in range(m, _TOPK):
                lo = jnp.minimum(s[j], t)
                if j < _TOPK - 1:
                    t = jnp.maximum(s[j], t)
                s[j] = lo
    for j in range(_TOPK):
        s_ref[:, j * _LANES:(j + 1) * _LANES] = s[j]


def _loss_kernel(xp_ref, xt_ref, xtt_ref, out_ref, s_ref):
    i = pl.program_id(0)

    @pl.when(i == 0)
    def _init():
        s_ref[...] = jnp.full((_Q, _CAND), jnp.inf, dtype=jnp.float32)

    xp = xp_ref[...]

    @pl.when(i < _NBF)
    def _main():
        _fold(s_ref, _dist2(xp, xt_ref[...], _KB), _KB // _LANES)

    @pl.when(i == _NBF)
    def _finish():
        d2t = _dist2(xp, xtt_ref[...], _TPAD)
        col = lax.broadcasted_iota(jnp.int32, (_Q, _TPAD), 1)
        d2t = jnp.where(col < _TAIL, d2t, jnp.inf)
        _fold(s_ref, d2t, _TPAD // _LANES)
        cand = s_ref[...]
        iota = lax.broadcasted_iota(jnp.int32, (_Q, _CAND), 1)
        vals = []
        for _ in range(_TOPK):
            m = jnp.min(cand, axis=1, keepdims=True)
            vals.append(m)
            hit = jnp.where(cand <= m, iota, _CAND)
            first = jnp.min(hit, axis=1, keepdims=True)
            cand = jnp.where(iota == first, jnp.inf, cand)
        d = jnp.sqrt(jnp.maximum(jnp.concatenate(vals, axis=1), 0.0))
        hinged = jnp.maximum(d - 1.0, 0.0)
        out_ref[...] = (jnp.sum(hinged) / (_Q * _TOPK)).reshape(1, 1)


def kernel(x_pred, x_target, top_k):
    out = pl.pallas_call(
        _loss_kernel,
        grid=(_NBF + 1,),
        in_specs=[
            pl.BlockSpec((_Q, _D), lambda i: (0, 0)),
            pl.BlockSpec((_KB, _D), lambda i: (jnp.minimum(i, _NBF - 1), 0)),
            pl.BlockSpec((_TPAD, _D),
                         lambda i: (jnp.where(i == _NBF, _TBLK, 0), 0)),
        ],
        out_specs=pl.BlockSpec((1, 1), lambda i: (0, 0)),
        out_shape=jax.ShapeDtypeStruct((1, 1), jnp.float32),
        scratch_shapes=[pltpu.VMEM((_Q, _CAND), jnp.float32)],
        compiler_params=pltpu.CompilerParams(
            dimension_semantics=("arbitrary",)),
    )(x_pred, x_target, x_target)
    return out[0, 0] + 0.0 * top_k


# speedup vs baseline: 1.2920x; 1.0182x over previous
import jax
import jax.numpy as jnp
from jax import lax
from jax.experimental import pallas as pl
from jax.experimental.pallas import tpu as pltpu

_Q = 1024
_D = 16
_K = 100000
_KB = 8192
_NBF = 12
_TAIL = _K - _NBF * _KB
_TPAD = 2048
_TBLK = _NBF * _KB // _TPAD
_TOPK = 5
_LANES = 128
_CAND = _TOPK * _LANES

_SORT8_BOTTOM5 = [
    (0, 1, True), (2, 3, True), (4, 5, True), (6, 7, True),
    (0, 2, True), (1, 3, True), (4, 6, True), (5, 7, True),
    (1, 2, True), (5, 6, True),
    (0, 4, True), (1, 5, True), (2, 6, False), (3, 7, False),
    (2, 4, True), (3, 5, False),
    (1, 2, True), (3, 4, True),
]


def _dist2(xp, xt, width):
    sq_p = jnp.sum(xp * xp, axis=1, keepdims=True)
    sq_t = jnp.sum(xt * xt, axis=1, keepdims=True)
    xp_aug = jnp.concatenate(
        [xp, sq_p, jnp.ones((_Q, 1), jnp.float32)], axis=1)
    xt_aug = jnp.concatenate(
        [-2.0 * xt, jnp.ones((width, 1), jnp.float32), sq_t], axis=1)
    return lax.dot_general(xp_aug, xt_aug, (((1,), (1,)), ((), ())),
                           preferred_element_type=jnp.float32)


def _fold(s_ref, d2, nch):
    s = [s_ref[:, j * _LANES:(j + 1) * _LANES] for j in range(_TOPK)]
    for g in range(nch // 8):
        v = [d2[:, (8 * g + c) * _LANES:(8 * g + c + 1) * _LANES]
             for c in range(8)]
        for i, j, need_max in _SORT8_BOTTOM5:
            lo = jnp.minimum(v[i], v[j])
            if need_max:
                v[j] = jnp.maximum(v[i], v[j])
            v[i] = lo
        for m in range(_TOPK):
            t = v[m]
            for j in range(m, _TOPK):
                lo = jnp.minimum(s[j], t)
                if j < _TOPK - 1:
                    t = jnp.maximum(s[j], t)
                s[j] = lo
    for j in range(_TOPK):
        s_ref[:, j * _LANES:(j + 1) * _LANES] = s[j]


def _loss_kernel(xp_ref, xt_ref, xtt_ref, out_ref, s_ref):
    i = pl.program_id(0)

    @pl.when(i == 0)
    def _init():
        s_ref[...] = jnp.full((_Q, _CAND), jnp.inf, dtype=jnp.float32)

    xp = xp_ref[...]

    @pl.when(i < _NBF)
    def _main():
        _fold(s_ref, _dist2(xp, xt_ref[...], _KB), _KB // _LANES)

    @pl.when(i == _NBF)
    def _finish():
        d2t = _dist2(xp, xtt_ref[...], _TPAD)
        col = lax.broadcasted_iota(jnp.int32, (_Q, _TPAD), 1)
        d2t = jnp.where(col < _TAIL, d2t, jnp.inf)
        _fold(s_ref, d2t, _TPAD // _LANES)
        cand = s_ref[...]
        iota = lax.broadcasted_iota(jnp.int32, (_Q, _CAND), 1)
        vals = []
        for _ in range(_TOPK):
            m = jnp.min(cand, axis=1, keepdims=True)
            vals.append(m)
            hit = jnp.where(cand <= m, iota, _CAND)
            first = jnp.min(hit, axis=1, keepdims=True)
            cand = jnp.where(iota == first, jnp.inf, cand)
        d = jnp.sqrt(jnp.maximum(jnp.concatenate(vals, axis=1), 0.0))
        hinged = jnp.maximum(d - 1.0, 0.0)
        out_ref[...] = (jnp.sum(hinged) / (_Q * _TOPK)).reshape(1, 1)


def kernel(x_pred, x_target, top_k):
    out = pl.pallas_call(
        _loss_kernel,
        grid=(_NBF + 1,),
        in_specs=[
            pl.BlockSpec((_Q, _D), lambda i: (0, 0)),
            pl.BlockSpec((_KB, _D), lambda i: (jnp.minimum(i, _NBF - 1), 0)),
            pl.BlockSpec((_TPAD, _D),
                         lambda i: (jnp.where(i == _NBF, _TBLK, 0), 0)),
        ],
        out_specs=pl.BlockSpec((1, 1), lambda i: (0, 0)),
        out_shape=jax.ShapeDtypeStruct((1, 1), jnp.float32),
        scratch_shapes=[pltpu.VMEM((_Q, _CAND), jnp.float32)],
        compiler_params=pltpu.CompilerParams(
            dimension_semantics=("arbitrary",)),
    )(x_pred, x_target, x_target)
    return out[0, 0] + 0.0 * top_k


# speedup vs baseline: 1.2972x; 1.0040x over previous
import jax
import jax.numpy as jnp
from jax import lax
from jax.experimental import pallas as pl
from jax.experimental.pallas import tpu as pltpu

_Q = 1024
_D = 16
_K = 100000
_KB = 8192
_NBF = 12
_TAIL = _K - _NBF * _KB
_TPAD = 2048
_TBLK = _NBF * _KB // _TPAD
_TOPK = 5
_LANES = 128
_CAND = _TOPK * _LANES

_SORT8_BOTTOM5 = [
    (0, 1, True), (2, 3, True), (4, 5, True), (6, 7, True),
    (0, 2, True), (1, 3, True), (4, 6, True), (5, 7, True),
    (1, 2, True), (5, 6, True),
    (0, 4, True), (1, 5, True), (2, 6, False), (3, 7, False),
    (2, 4, True), (3, 5, False),
    (1, 2, True), (3, 4, True),
]


def _dist2(xp, xt, width):
    sq_p = jnp.sum(xp * xp, axis=1, keepdims=True)
    sq_t = jnp.sum(xt * xt, axis=1, keepdims=True)
    xp_aug = jnp.concatenate(
        [xp, sq_p, jnp.ones((_Q, 1), jnp.float32)], axis=1)
    xt_aug = jnp.concatenate(
        [-2.0 * xt, jnp.ones((width, 1), jnp.float32), sq_t], axis=1)
    return lax.dot_general(xp_aug, xt_aug, (((1,), (1,)), ((), ())),
                           preferred_element_type=jnp.float32)


def _fold(s_ref, d2, nch):
    s = [s_ref[:, j * _LANES:(j + 1) * _LANES] for j in range(_TOPK)]
    for g in range(nch // 8):
        v = [d2[:, (8 * g + c) * _LANES:(8 * g + c + 1) * _LANES]
             for c in range(8)]
        for i, j, need_max in _SORT8_BOTTOM5:
            lo = jnp.minimum(v[i], v[j])
            if need_max:
                v[j] = jnp.maximum(v[i], v[j])
            v[i] = lo
        for m in range(_TOPK):
            t = v[m]
            for j in range(m, _TOPK):
                lo = jnp.minimum(s[j], t)
                if j < _TOPK - 1:
                    t = jnp.maximum(s[j], t)
                s[j] = lo
    for j in range(_TOPK):
        s_ref[:, j * _LANES:(j + 1) * _LANES] = s[j]


def _loss_kernel(xp_ref, xt_ref, xtt_ref, out_ref, s_ref):
    i = pl.program_id(0)

    @pl.when(i == 0)
    def _init():
        s_ref[...] = jnp.full((_Q, _CAND), jnp.inf, dtype=jnp.float32)

    xp = xp_ref[...]

    @pl.when(i < _NBF)
    def _main():
        _fold(s_ref, _dist2(xp, xt_ref[...], _KB), _KB // _LANES)

    @pl.when(i == _NBF)
    def _finish():
        d2t = _dist2(xp, xtt_ref[...], _TPAD)
        col = lax.broadcasted_iota(jnp.int32, (_Q, _TPAD), 1)
        d2t = jnp.where(col < _TAIL, d2t, jnp.inf)
        _fold(s_ref, d2t, _TPAD // _LANES)
        cand = s_ref[...]
        iota = lax.broadcasted_iota(jnp.int32, (_Q, _CAND), 1)
        vals = []
        for _ in range(_TOPK):
            m = jnp.min(cand, axis=1, keepdims=True)
            vals.append(m)
            hit = jnp.where(cand <= m, iota, _CAND)
            first = jnp.min(hit, axis=1, keepdims=True)
            cand = jnp.where(iota == first, jnp.inf, cand)
        d = jnp.sqrt(jnp.maximum(jnp.concatenate(vals, axis=1), 0.0))
        hinged = jnp.maximum(d - 1.0, 0.0)
        out_ref[...] = (jnp.sum(hinged) / (_Q * _TOPK)).reshape(1, 1)


def kernel(x_pred, x_target, top_k):
    out = pl.pallas_call(
        _loss_kernel,
        grid=(_NBF + 1,),
        in_specs=[
            pl.BlockSpec((_Q, _D), lambda i: (0, 0)),
            pl.BlockSpec((_KB, _D), lambda i: (jnp.minimum(i, _NBF - 1), 0)),
            pl.BlockSpec((_TPAD, _D),
                         lambda i: (jnp.where(i == _NBF, _TBLK, 0), 0)),
        ],
        out_specs=pl.BlockSpec((1, 1), lambda i: (0, 0)),
        out_shape=jax.ShapeDtypeStruct((1, 1), jnp.float32),
        scratch_shapes=[pltpu.VMEM((_Q, _CAND), jnp.float32)],
        compiler_params=pltpu.CompilerParams(
            dimension_semantics=("arbitrary",)),
    )(x_pred, x_target, x_target)
    del top_k
    return out[0, 0]


# speedup vs baseline: 1.4154x; 1.0911x over previous
import jax
import jax.numpy as jnp
from jax import lax
from jax.experimental import pallas as pl
from jax.experimental.pallas import tpu as pltpu

_Q = 1024
_D = 16
_K = 100000
_KB = 8192
_NBF = 12
_TAIL = _K - _NBF * _KB
_TPAD = 2048
_TBLK = _NBF * _KB // _TPAD
_TOPK = 5
_LANES = 128
_CAND = _TOPK * _LANES

_SORT8_BOTTOM5 = [
    (0, 1, True), (2, 3, True), (4, 5, True), (6, 7, True),
    (0, 2, True), (1, 3, True), (4, 6, True), (5, 7, True),
    (1, 2, True), (5, 6, True),
    (0, 4, True), (1, 5, True), (2, 6, False), (3, 7, False),
    (2, 4, True), (3, 5, False),
    (1, 2, True), (3, 4, True),
]


def _dist2(xp, xt, width):
    sq_p = jnp.sum(xp * xp, axis=1, keepdims=True)
    sq_t = jnp.sum(xt * xt, axis=1, keepdims=True)
    xp_aug = jnp.concatenate(
        [xp, sq_p, jnp.ones((_Q, 1), jnp.float32)], axis=1)
    xt_aug = jnp.concatenate(
        [-2.0 * xt, jnp.ones((width, 1), jnp.float32), sq_t], axis=1)
    return lax.dot_general(xp_aug, xt_aug, (((1,), (1,)), ((), ())),
                           preferred_element_type=jnp.float32)


def _sort8_b5(v):
    v = list(v)
    for i, j, need_max in _SORT8_BOTTOM5:
        lo = jnp.minimum(v[i], v[j])
        if need_max:
            v[j] = jnp.maximum(v[i], v[j])
        v[i] = lo
    return v[:5]


def _merge55(pa, pb):
    lm = [jnp.minimum(pa[i], pb[4 - i]) for i in range(5)]
    a = jnp.minimum(lm[0], lm[4])
    b = jnp.maximum(lm[0], lm[4])
    c = jnp.minimum(b, lm[2])
    d = jnp.maximum(b, lm[2])
    e = jnp.minimum(lm[1], lm[3])
    f = jnp.maximum(lm[1], lm[3])
    return [a, jnp.minimum(c, e), jnp.maximum(c, e),
            jnp.minimum(d, f), jnp.maximum(d, f)]


def _group_b5(v):
    if len(v) == 8:
        return _sort8_b5(v)
    h = len(v) // 2
    return _merge55(_group_b5(v[:h]), _group_b5(v[h:]))


def _fold(s_ref, d2, nch):
    s = [s_ref[:, j * _LANES:(j + 1) * _LANES] for j in range(_TOPK)]
    done = 0
    while done < nch:
        g = 32 if nch - done >= 32 else (16 if nch - done >= 16 else 8)
        v = [d2[:, (done + c) * _LANES:(done + c + 1) * _LANES]
             for c in range(g)]
        done += g
        b5 = _group_b5(v)
        for m in range(_TOPK):
            t = b5[m]
            for j in range(m, _TOPK):
                lo = jnp.minimum(s[j], t)
                if j < _TOPK - 1:
                    t = jnp.maximum(s[j], t)
                s[j] = lo
    for j in range(_TOPK):
        s_ref[:, j * _LANES:(j + 1) * _LANES] = s[j]


def _loss_kernel(xp_ref, xt_ref, xtt_ref, out_ref, s_ref):
    i = pl.program_id(0)

    @pl.when(i == 0)
    def _init():
        s_ref[...] = jnp.full((_Q, _CAND), jnp.inf, dtype=jnp.float32)

    xp = xp_ref[...]

    @pl.when(i < _NBF)
    def _main():
        _fold(s_ref, _dist2(xp, xt_ref[...], _KB), _KB // _LANES)

    @pl.when(i == _NBF)
    def _finish():
        d2t = _dist2(xp, xtt_ref[...], _TPAD)
        col = lax.broadcasted_iota(jnp.int32, (_Q, _TPAD), 1)
        d2t = jnp.where(col < _TAIL, d2t, jnp.inf)
        _fold(s_ref, d2t, _TPAD // _LANES)
        cand = s_ref[...]
        iota = lax.broadcasted_iota(jnp.int32, (_Q, _CAND), 1)
        vals = []
        for _ in range(_TOPK):
            m = jnp.min(cand, axis=1, keepdims=True)
            vals.append(m)
            hit = jnp.where(cand <= m, iota, _CAND)
            first = jnp.min(hit, axis=1, keepdims=True)
            cand = jnp.where(iota == first, jnp.inf, cand)
        d = jnp.sqrt(jnp.maximum(jnp.concatenate(vals, axis=1), 0.0))
        hinged = jnp.maximum(d - 1.0, 0.0)
        out_ref[...] = (jnp.sum(hinged) / (_Q * _TOPK)).reshape(1, 1)


def kernel(x_pred, x_target, top_k):
    out = pl.pallas_call(
        _loss_kernel,
        grid=(_NBF + 1,),
        in_specs=[
            pl.BlockSpec((_Q, _D), lambda i: (0, 0)),
            pl.BlockSpec((_KB, _D), lambda i: (jnp.minimum(i, _NBF - 1), 0)),
            pl.BlockSpec((_TPAD, _D),
                         lambda i: (jnp.where(i == _NBF, _TBLK, 0), 0)),
        ],
        out_specs=pl.BlockSpec((1, 1), lambda i: (0, 0)),
        out_shape=jax.ShapeDtypeStruct((1, 1), jnp.float32),
        scratch_shapes=[pltpu.VMEM((_Q, _CAND), jnp.float32)],
        compiler_params=pltpu.CompilerParams(
            dimension_semantics=("arbitrary",)),
    )(x_pred, x_target, x_target)
    del top_k
    return out[0, 0]


# speedup vs baseline: 1.4416x; 1.0185x over previous
import jax
import jax.numpy as jnp
from jax import lax
from jax.experimental import pallas as pl
from jax.experimental.pallas import tpu as pltpu

_Q = 1024
_D = 16
_K = 100000
_KB = 8192
_NBF = 12
_TAIL = _K - _NBF * _KB
_TPAD = 2048
_TBLK = _NBF * _KB // _TPAD
_TOPK = 5
_LANES = 128
_CAND = _TOPK * _LANES

_SORT8_BOTTOM5 = [
    (0, 1, True), (2, 3, True), (4, 5, True), (6, 7, True),
    (0, 2, True), (1, 3, True), (4, 6, True), (5, 7, True),
    (1, 2, True), (5, 6, True),
    (0, 4, True), (1, 5, True), (2, 6, False), (3, 7, False),
    (2, 4, True), (3, 5, False),
    (1, 2, True), (3, 4, True),
]


def _dist2(xp, xt, width):
    sq_p = jnp.sum(xp * xp, axis=1, keepdims=True)
    sq_t = jnp.sum(xt * xt, axis=1, keepdims=True)
    xp_aug = jnp.concatenate(
        [xp, sq_p, jnp.ones((_Q, 1), jnp.float32)], axis=1)
    xt_aug = jnp.concatenate(
        [-2.0 * xt, jnp.ones((width, 1), jnp.float32), sq_t], axis=1)
    return lax.dot_general(xp_aug, xt_aug, (((1,), (1,)), ((), ())),
                           preferred_element_type=jnp.float32)


def _sort8_b5(v):
    v = list(v)
    for i, j, need_max in _SORT8_BOTTOM5:
        lo = jnp.minimum(v[i], v[j])
        if need_max:
            v[j] = jnp.maximum(v[i], v[j])
        v[i] = lo
    return v[:5]


def _merge55(pa, pb):
    lm = [jnp.minimum(pa[i], pb[4 - i]) for i in range(5)]
    a = jnp.minimum(lm[0], lm[4])
    b = jnp.maximum(lm[0], lm[4])
    c = jnp.minimum(b, lm[2])
    d = jnp.maximum(b, lm[2])
    e = jnp.minimum(lm[1], lm[3])
    f = jnp.maximum(lm[1], lm[3])
    return [a, jnp.minimum(c, e), jnp.maximum(c, e),
            jnp.minimum(d, f), jnp.maximum(d, f)]


def _group_b5(v):
    if len(v) == 8:
        return _sort8_b5(v)
    h = len(v) // 2
    return _merge55(_group_b5(v[:h]), _group_b5(v[h:]))


def _fold(s_ref, d2, nch):
    s = [s_ref[:, j * _LANES:(j + 1) * _LANES] for j in range(_TOPK)]
    done = 0
    while done < nch:
        g = 32 if nch - done >= 32 else (16 if nch - done >= 16 else 8)
        v = [d2[:, (done + c) * _LANES:(done + c + 1) * _LANES]
             for c in range(g)]
        done += g
        s = _merge55(s, _group_b5(v))
    for j in range(_TOPK):
        s_ref[:, j * _LANES:(j + 1) * _LANES] = s[j]


def _loss_kernel(xp_ref, xt_ref, xtt_ref, out_ref, s_ref):
    i = pl.program_id(0)

    @pl.when(i == 0)
    def _init():
        s_ref[...] = jnp.full((_Q, _CAND), jnp.inf, dtype=jnp.float32)

    xp = xp_ref[...]

    @pl.when(i < _NBF)
    def _main():
        _fold(s_ref, _dist2(xp, xt_ref[...], _KB), _KB // _LANES)

    @pl.when(i == _NBF)
    def _finish():
        d2t = _dist2(xp, xtt_ref[...], _TPAD)
        col = lax.broadcasted_iota(jnp.int32, (_Q, _TPAD), 1)
        d2t = jnp.where(col < _TAIL, d2t, jnp.inf)
        _fold(s_ref, d2t, _TPAD // _LANES)
        cand = s_ref[...]
        iota = lax.broadcasted_iota(jnp.int32, (_Q, _CAND), 1)
        vals = []
        for _ in range(_TOPK):
            m = jnp.min(cand, axis=1, keepdims=True)
            vals.append(m)
            hit = jnp.where(cand <= m, iota, _CAND)
            first = jnp.min(hit, axis=1, keepdims=True)
            cand = jnp.where(iota == first, jnp.inf, cand)
        d = jnp.sqrt(jnp.maximum(jnp.concatenate(vals, axis=1), 0.0))
        hinged = jnp.maximum(d - 1.0, 0.0)
        out_ref[...] = (jnp.sum(hinged) / (_Q * _TOPK)).reshape(1, 1)


def kernel(x_pred, x_target, top_k):
    out = pl.pallas_call(
        _loss_kernel,
        grid=(_NBF + 1,),
        in_specs=[
            pl.BlockSpec((_Q, _D), lambda i: (0, 0)),
            pl.BlockSpec((_KB, _D), lambda i: (jnp.minimum(i, _NBF - 1), 0)),
            pl.BlockSpec((_TPAD, _D),
                         lambda i: (jnp.where(i == _NBF, _TBLK, 0), 0)),
        ],
        out_specs=pl.BlockSpec((1, 1), lambda i: (0, 0)),
        out_shape=jax.ShapeDtypeStruct((1, 1), jnp.float32),
        scratch_shapes=[pltpu.VMEM((_Q, _CAND), jnp.float32)],
        compiler_params=pltpu.CompilerParams(
            dimension_semantics=("arbitrary",)),
    )(x_pred, x_target, x_target)
    del top_k
    return out[0, 0]
